# baseline (speedup 1.0000x reference)
import functools

import jax
import jax.numpy as jnp
from jax import lax
from jax.experimental import pallas as pl
from jax.experimental.pallas import tpu as pltpu
from jax.experimental.pallas import tpu_sc as plsc

OUT_DIM = 128
BATCH = 1024
SEQ = 300

NUM_CORES = 2
NUM_SUBCORES = 16
NW = NUM_CORES * NUM_SUBCORES

B = BATCH * SEQ
B_PER_W = B // NW
CHUNK = 128
NCH = B_PER_W // CHUNK
NPAIR = NCH // 2


def _gather_rows2(ids_flat, table):
    mesh = plsc.VectorSubcoreMesh(
        core_axis_name="c", subcore_axis_name="s",
        num_cores=NUM_CORES, num_subcores=NUM_SUBCORES)

    out_sds = jax.ShapeDtypeStruct((B, OUT_DIM), jnp.float32)

    NBUF = 4

    @functools.partial(
        pl.kernel,
        out_type=(out_sds, out_sds),
        mesh=mesh,
        scratch_types=[
            pltpu.VMEM((B_PER_W,), jnp.int32),
        ] + [pltpu.VMEM((CHUNK, OUT_DIM), jnp.float32)] * NBUF
          + [pltpu.SemaphoreType.DMA] * (2 * NBUF),
    )
    def k(ids_hbm, table_hbm, out_hbm, out2_hbm, idx_v, *bufs_sems):
        bufs = bufs_sems[:NBUF]
        gsem = bufs_sems[NBUF:2 * NBUF]
        ssem = bufs_sems[2 * NBUF:]
        wid = lax.axis_index("s") * NUM_CORES + lax.axis_index("c")
        base = pl.multiple_of(wid * B_PER_W, CHUNK)
        pltpu.sync_copy(ids_hbm.at[pl.ds(base, B_PER_W)], idx_v)

        def start_gather(c, j):
            off = pl.multiple_of(c * CHUNK, CHUNK)
            pltpu.async_copy(table_hbm.at[idx_v.at[pl.ds(off, CHUNK)]],
                             bufs[j], gsem[j])

        def wait_gather(j):
            pltpu.make_async_copy(table_hbm.at[pl.ds(0, CHUNK)],
                                  bufs[j], gsem[j]).wait()

        def start_stores(c, j):
            off = pl.multiple_of(base + c * CHUNK, CHUNK)
            pltpu.async_copy(bufs[j], out_hbm.at[pl.ds(off, CHUNK)], ssem[j])
            pltpu.async_copy(bufs[j], out2_hbm.at[pl.ds(off, CHUNK)], ssem[j])

        def wait_stores(j):
            for _ in range(2):
                pltpu.make_async_copy(bufs[j], out_hbm.at[pl.ds(0, CHUNK)],
                                      ssem[j]).wait()

        DEPTH = NBUF - 1
        for c in range(DEPTH):
            start_gather(c, c % NBUF)
        for c in range(NCH):
            j = c % NBUF
            wait_gather(j)
            start_stores(c, j)
            nc = c + DEPTH
            if nc < NCH:
                jn = nc % NBUF
                if nc >= NBUF:
                    wait_stores(jn)
                start_gather(nc, jn)
        for c in range(NCH - NBUF, NCH):
            wait_stores(c % NBUF)

    return k(ids_flat, table)


def kernel(input_ids, attention_mask, W):
    ids_sm = input_ids.T.reshape(-1)
    f1, f2 = _gather_rows2(ids_sm, W)
    e1 = f1.reshape(SEQ, BATCH, OUT_DIM).transpose(1, 0, 2)
    e2 = f2.reshape(SEQ, BATCH, OUT_DIM).transpose(1, 0, 2)
    return (e1, e2, attention_mask)


# speedup vs baseline: 1.0007x; 1.0007x over previous
import functools

import jax
import jax.numpy as jnp
from jax import lax
from jax.experimental import pallas as pl
from jax.experimental.pallas import tpu as pltpu
from jax.experimental.pallas import tpu_sc as plsc

OUT_DIM = 128
BATCH = 1024
SEQ = 300

NUM_CORES = 2
NUM_SUBCORES = 16
NW = NUM_CORES * NUM_SUBCORES

B = BATCH * SEQ
B_PER_W = B // NW
CHUNK = 128
NCH = B_PER_W // CHUNK
NPAIR = NCH // 2


def _gather_rows2(ids_flat, table):
    mesh = plsc.VectorSubcoreMesh(
        core_axis_name="c", subcore_axis_name="s",
        num_cores=NUM_CORES, num_subcores=NUM_SUBCORES)

    out_sds = jax.ShapeDtypeStruct((B, OUT_DIM), jnp.float32)

    @functools.partial(
        pl.kernel,
        out_type=(out_sds, out_sds),
        mesh=mesh,
        scratch_types=[
            pltpu.VMEM((B_PER_W,), jnp.int32),
            pltpu.VMEM((CHUNK, OUT_DIM), jnp.float32),
            pltpu.VMEM((CHUNK, OUT_DIM), jnp.float32),
            pltpu.SemaphoreType.DMA,
            pltpu.SemaphoreType.DMA,
        ],
    )
    def k(ids_hbm, table_hbm, out_hbm, out2_hbm, idx_v, buf0, buf1, sem0, sem1):
        wid = lax.axis_index("s") * NUM_CORES + lax.axis_index("c")
        base = pl.multiple_of(wid * B_PER_W, CHUNK)
        pltpu.sync_copy(ids_hbm.at[pl.ds(base, B_PER_W)], idx_v)

        def start_gather(c, buf, sem):
            off = pl.multiple_of(c * CHUNK, CHUNK)
            pltpu.async_copy(table_hbm.at[idx_v.at[pl.ds(off, CHUNK)]], buf, sem)

        def wait_gather(buf, sem):
            pltpu.make_async_copy(table_hbm.at[pl.ds(0, CHUNK)], buf, sem).wait()

        def store(c, buf):
            off = pl.multiple_of(base + c * CHUNK, CHUNK)
            pltpu.sync_copy(buf, out_hbm.at[pl.ds(off, CHUNK)])
            pltpu.sync_copy(buf, out2_hbm.at[pl.ds(off, CHUNK)])

        start_gather(0, buf0, sem0)

        @pl.loop(0, NPAIR)
        def _(i):
            c0 = 2 * i
            start_gather(c0 + 1, buf1, sem1)
            wait_gather(buf0, sem0)
            store(c0, buf0)
            start_gather(c0 + 2, buf0, sem0)
            wait_gather(buf1, sem1)
            store(c0 + 1, buf1)

        wait_gather(buf0, sem0)
        store(NCH - 1, buf0)

    return k(ids_flat, table)


def kernel(input_ids, attention_mask, W):
    ids_sm = input_ids.T.reshape(-1)
    f1, f2 = _gather_rows2(ids_sm, W)
    e1 = f1.reshape(SEQ, BATCH, OUT_DIM).transpose(1, 0, 2)
    e2 = f2.reshape(SEQ, BATCH, OUT_DIM).transpose(1, 0, 2)
    return (e1, e2, attention_mask)
